# Initial kernel scaffold; baseline (speedup 1.0000x reference)
import jax
import jax.numpy as jnp
from jax.experimental import pallas as pl


def kernel(x, edge_index, batch, W0, b0, gn0_w, gn0_b, gn0_a, ggc_w, gru_wih, gru_whh, gru_bih, gru_bhh, gn1_w, gn1_b, gn1_a, W_lin, b_lin):
    raise NotImplementedError("write your pallas kernel here")


# speedup vs baseline: 4.8690x; 4.8690x over previous
import functools

import jax
import jax.numpy as jnp
from jax import lax
from jax.experimental import pallas as pl
from jax.experimental.pallas import tpu as pltpu
from jax.experimental.pallas import tpu_sc as plsc

N = 10000
E = 320000
D = 128
G = 16

NP = 10240
BN = 1024
NB = NP // BN
NC = 2
NS = 16
NW = NC * NS
K = 128
EW = 10240
NCH = EW // K
EP = NW * EW
RPT = NP // NS
DW = 16
EPS = 1e-5
HI = lax.Precision.HIGHEST


def _sc_scatter_body(y_hbm, row_hbm, col_hbm, zeros_hbm, z_hbm,
                     row_v, col_v, buf, z_sh, sem):
    c = lax.axis_index("c")
    s = lax.axis_index("s")
    wid = s * NC + c
    pltpu.sync_copy(row_hbm.at[wid], row_v)
    pltpu.sync_copy(col_hbm.at[wid], col_v)
    pltpu.sync_copy(zeros_hbm.at[pl.ds(s * RPT, RPT)],
                    z_sh.at[pl.ds(s * RPT, RPT)])
    plsc.subcore_barrier()

    @pl.loop(0, NCH)
    def _(j):
        pltpu.async_copy(y_hbm.at[row_v.at[j]], buf, sem).wait()
        pltpu.sync_copy(buf, z_sh.at[col_v.at[j]], add=True)

    plsc.subcore_barrier()
    pltpu.sync_copy(z_sh.at[pl.ds(s * RPT, RPT)],
                    z_hbm.at[c, pl.ds(s * RPT, RPT)])


@functools.lru_cache(maxsize=None)
def _sc_scatter():
    return pl.kernel(
        _sc_scatter_body,
        out_type=jax.ShapeDtypeStruct((NC, NP, D), jnp.float32),
        mesh=plsc.VectorSubcoreMesh(core_axis_name="c", subcore_axis_name="s",
                                    num_cores=NC, num_subcores=NS),
        scratch_types=[
            pltpu.VMEM((NCH, K), jnp.int32),
            pltpu.VMEM((NCH, K), jnp.int32),
            pltpu.VMEM((K, D), jnp.float32),
            pltpu.VMEM_SHARED((NP, D), jnp.float32),
            pltpu.SemaphoreType.DMA,
        ],
    )


def _sc_deg_body(col_hbm, zeros_hbm, ones_hbm, deg_hbm,
                 col_v, ones_v, deg_sh):
    c = lax.axis_index("c")
    s = lax.axis_index("s")
    wid = s * NC + c
    pltpu.sync_copy(col_hbm.at[wid], col_v)
    pltpu.sync_copy(ones_hbm, ones_v)
    pltpu.sync_copy(zeros_hbm.at[pl.ds(s * RPT, RPT)],
                    deg_sh.at[pl.ds(s * RPT, RPT)])
    plsc.subcore_barrier()

    @pl.loop(0, NCH)
    def _(j):
        pltpu.sync_copy(ones_v, deg_sh.at[col_v.at[j]], add=True)

    plsc.subcore_barrier()
    pltpu.sync_copy(deg_sh.at[pl.ds(s * RPT, RPT)],
                    deg_hbm.at[c, pl.ds(s * RPT, RPT)])


@functools.lru_cache(maxsize=None)
def _sc_deg():
    return pl.kernel(
        _sc_deg_body,
        out_type=jax.ShapeDtypeStruct((NC, NP, DW), jnp.float32),
        mesh=plsc.VectorSubcoreMesh(core_axis_name="c", subcore_axis_name="s",
                                    num_cores=NC, num_subcores=NS),
        scratch_types=[
            pltpu.VMEM((NCH, K), jnp.int32),
            pltpu.VMEM((K, DW), jnp.float32),
            pltpu.VMEM_SHARED((NP, DW), jnp.float32),
        ],
    )


def _dinv(deg_ref):
    deg = deg_ref[0, :, 0] + deg_ref[1, :, 0] + 1.0
    return lax.rsqrt(deg)


def _tc_y_body(x_ref, w_ref, deg_ref, y_ref):
    xw = jnp.dot(x_ref[...], w_ref[...], preferred_element_type=jnp.float32,
                 precision=HI)
    y_ref[...] = xw * _dinv(deg_ref)[:, None]


def _seg_stats(bvec, t):
    oh = (bvec[None, :] == lax.broadcasted_iota(jnp.int32, (G, BN), 0))
    oh = oh.astype(jnp.float32)
    s1 = jnp.dot(oh, t, preferred_element_type=jnp.float32, precision=HI)
    s2 = jnp.dot(oh, t * t, preferred_element_type=jnp.float32, precision=HI)
    cnt = jnp.broadcast_to(jnp.sum(oh, axis=1)[:, None], (G, D))
    return s1, s2, cnt


def _tc_t_body(z_ref, y_ref, deg_ref, b0_ref, batch_ref, t_ref, stats_ref):
    i = pl.program_id(0)
    dinv = _dinv(deg_ref)
    t = dinv[:, None] * (z_ref[0] + z_ref[1] + y_ref[...]) + b0_ref[...]
    t_ref[...] = t
    s1, s2, cnt = _seg_stats(batch_ref[0, :], t)
    blk = jnp.stack([s1, s2, cnt])

    @pl.when(i == 0)
    def _():
        stats_ref[...] = blk

    @pl.when(i > 0)
    def _():
        stats_ref[...] = stats_ref[...] + blk


def _norm_factors(s1, s2, cnt, a):
    cnt = jnp.maximum(cnt, 1.0)
    mean = s1 / cnt
    var = s2 / cnt - (2.0 * a) * mean * mean + (a * a) * mean * mean
    invstd = lax.rsqrt(jnp.maximum(var, 0.0) + EPS)
    return mean * a, invstd


def _row_norm(bvec, t, mean_a, invstd, w, b):
    oh = (bvec[:, None] == lax.broadcasted_iota(jnp.int32, (BN, G), 1))
    oh = oh.astype(jnp.float32)
    mean_rows = jnp.dot(oh, mean_a, preferred_element_type=jnp.float32,
                        precision=HI)
    invstd_rows = jnp.dot(oh, invstd, preferred_element_type=jnp.float32,
                          precision=HI)
    return w * ((t - mean_rows) * invstd_rows) + b


def _tc_h0_body(t_ref, stats_ref, batch_ref, w_ref, b_ref, a_ref, ggw_ref,
                h_ref, m_ref):
    st = stats_ref[...]
    mean_a, invstd = _norm_factors(st[0], st[1], st[2], a_ref[...])
    h0 = _row_norm(batch_ref[0, :], t_ref[...], mean_a, invstd,
                   w_ref[...], b_ref[...])
    h_ref[...] = h0
    m_ref[...] = jnp.dot(h0, ggw_ref[...], preferred_element_type=jnp.float32,
                         precision=HI)


def _gru(z_ref, h_ref, wih_ref, whh_ref, bih_ref, bhh_ref):
    agg = z_ref[0] + z_ref[1]
    h = h_ref[...]
    gi = jnp.dot(agg, wih_ref[...], preferred_element_type=jnp.float32,
                 precision=HI) + bih_ref[...]
    gh = jnp.dot(h, whh_ref[...], preferred_element_type=jnp.float32,
                 precision=HI) + bhh_ref[...]
    r = jax.nn.sigmoid(gi[:, :D] + gh[:, :D])
    zz = jax.nn.sigmoid(gi[:, D:2 * D] + gh[:, D:2 * D])
    n = jnp.tanh(gi[:, 2 * D:] + r * gh[:, 2 * D:])
    return (1.0 - zz) * n + zz * h


def _tc_gru_body(z_ref, h_ref, wih_ref, whh_ref, bih_ref, bhh_ref, wn_ref,
                 hn_ref, m_ref):
    hn = _gru(z_ref, h_ref, wih_ref, whh_ref, bih_ref, bhh_ref)
    hn_ref[...] = hn
    m_ref[...] = jnp.dot(hn, wn_ref[...], preferred_element_type=jnp.float32,
                         precision=HI)


def _tc_gru2_body(z_ref, h_ref, wih_ref, whh_ref, bih_ref, bhh_ref, batch_ref,
                  u_ref, stats_ref):
    i = pl.program_id(0)
    hn = _gru(z_ref, h_ref, wih_ref, whh_ref, bih_ref, bhh_ref)
    u = 0.5 * hn * (1.0 + lax.erf(hn * 0.7071067811865476))
    u_ref[...] = u
    s1, s2, cnt = _seg_stats(batch_ref[0, :], u)
    blk = jnp.stack([s1, s2, cnt])

    @pl.when(i == 0)
    def _():
        stats_ref[...] = blk

    @pl.when(i > 0)
    def _():
        stats_ref[...] = stats_ref[...] + blk


def _tc_fin_body(u_ref, stats_ref, batch_ref, w_ref, b_ref, a_ref, h0_ref,
                 wlin_ref, blin_ref, out_ref):
    st = stats_ref[...]
    mean_a, invstd = _norm_factors(st[0], st[1], st[2], a_ref[...])
    gn = _row_norm(batch_ref[0, :], u_ref[...], mean_a, invstd,
                   w_ref[...], b_ref[...])
    v = h0_ref[...] + gn
    out_ref[...] = jnp.dot(v, wlin_ref[...],
                           preferred_element_type=jnp.float32,
                           precision=HI) + blin_ref[...]


_ROW = pl.BlockSpec((BN, D), lambda i: (i, 0))
_ROW3 = pl.BlockSpec((BN, 3 * D), lambda i: (i, 0))
_ZPART = pl.BlockSpec((NC, BN, D), lambda i: (0, i, 0))
_DEG = pl.BlockSpec((NC, BN, DW), lambda i: (0, i, 0))
_BATCH = pl.BlockSpec((1, BN), lambda i: (0, i))
_FULL = lambda *shape: pl.BlockSpec(shape, lambda i: (0,) * len(shape))

_f32 = jnp.float32


def _call_y(x_p, W0, deg):
    return pl.pallas_call(
        _tc_y_body,
        grid=(NB,),
        in_specs=[_ROW, _FULL(D, D), _DEG],
        out_specs=_ROW,
        out_shape=jax.ShapeDtypeStruct((NP, D), _f32),
    )(x_p, W0, deg)


def _call_t(z, y, deg, b0, batch_p):
    return pl.pallas_call(
        _tc_t_body,
        grid=(NB,),
        in_specs=[_ZPART, _ROW, _DEG, _FULL(1, D), _BATCH],
        out_specs=[_ROW, _FULL(3, G, D)],
        out_shape=[jax.ShapeDtypeStruct((NP, D), _f32),
                   jax.ShapeDtypeStruct((3, G, D), _f32)],
    )(z, y, deg, b0, batch_p)


def _call_h0(t, stats, batch_p, w, b, a, ggw):
    return pl.pallas_call(
        _tc_h0_body,
        grid=(NB,),
        in_specs=[_ROW, _FULL(3, G, D), _BATCH, _FULL(1, D), _FULL(1, D),
                  _FULL(1, D), _FULL(D, D)],
        out_specs=[_ROW, _ROW],
        out_shape=[jax.ShapeDtypeStruct((NP, D), _f32),
                   jax.ShapeDtypeStruct((NP, D), _f32)],
    )(t, stats, batch_p, w, b, a, ggw)


def _call_gru(z, h, wih_t, whh_t, bih, bhh, wn):
    return pl.pallas_call(
        _tc_gru_body,
        grid=(NB,),
        in_specs=[_ZPART, _ROW, _FULL(D, 3 * D), _FULL(D, 3 * D),
                  _FULL(1, 3 * D), _FULL(1, 3 * D), _FULL(D, D)],
        out_specs=[_ROW, _ROW],
        out_shape=[jax.ShapeDtypeStruct((NP, D), _f32),
                   jax.ShapeDtypeStruct((NP, D), _f32)],
    )(z, h, wih_t, whh_t, bih, bhh, wn)


def _call_gru2(z, h, wih_t, whh_t, bih, bhh, batch_p):
    return pl.pallas_call(
        _tc_gru2_body,
        grid=(NB,),
        in_specs=[_ZPART, _ROW, _FULL(D, 3 * D), _FULL(D, 3 * D),
                  _FULL(1, 3 * D), _FULL(1, 3 * D), _BATCH],
        out_specs=[_ROW, _FULL(3, G, D)],
        out_shape=[jax.ShapeDtypeStruct((NP, D), _f32),
                   jax.ShapeDtypeStruct((3, G, D), _f32)],
    )(z, h, wih_t, whh_t, bih, bhh, batch_p)


def _call_fin(u, stats2, batch_p, w, b, a, h0, wlin_t, blin):
    return pl.pallas_call(
        _tc_fin_body,
        grid=(NB,),
        in_specs=[_ROW, _FULL(3, G, D), _BATCH, _FULL(1, D), _FULL(1, D),
                  _FULL(1, D), _ROW, _FULL(D, D), _FULL(1, D)],
        out_specs=_ROW,
        out_shape=jax.ShapeDtypeStruct((NP, D), _f32),
    )(u, stats2, batch_p, w, b, a, h0, wlin_t, blin)


def kernel(x, edge_index, batch, W0, b0, gn0_w, gn0_b, gn0_a, ggc_w,
           gru_wih, gru_whh, gru_bih, gru_bhh, gn1_w, gn1_b, gn1_a,
           W_lin, b_lin):
    x_p = jnp.zeros((NP, D), _f32).at[:N].set(x)
    batch_p = jnp.full((1, NP), G, jnp.int32).at[0, :N].set(batch)

    pad = EP - E
    dummy = jnp.full((pad,), NP - 1, jnp.int32)
    row = jnp.concatenate([edge_index[0], dummy]).reshape(NW, NCH, K)
    col = jnp.concatenate([edge_index[1], dummy]).reshape(NW, NCH, K)

    zeros_d = jnp.zeros((NP, D), _f32)
    zeros_w = jnp.zeros((NP, DW), _f32)
    ones_w = jnp.ones((K, DW), _f32)

    r2 = lambda v: v.reshape(1, -1)
    wih_t = gru_wih.T
    whh_t = gru_whh.T

    deg = _sc_deg()(col, zeros_w, ones_w)
    y = _call_y(x_p, W0, deg)
    z = _sc_scatter()(y, row, col, zeros_d)
    t, stats = _call_t(z, y, deg, r2(b0), batch_p)
    h0, m0 = _call_h0(t, stats, batch_p, r2(gn0_w), r2(gn0_b), r2(gn0_a),
                      ggc_w[0])
    a0 = _sc_scatter()(m0, row, col, zeros_d)
    h1, m1 = _call_gru(a0, h0, wih_t, whh_t, r2(gru_bih), r2(gru_bhh),
                       ggc_w[1])
    a1 = _sc_scatter()(m1, row, col, zeros_d)
    u, stats2 = _call_gru2(a1, h1, wih_t, whh_t, r2(gru_bih), r2(gru_bhh),
                           batch_p)
    out = _call_fin(u, stats2, batch_p, r2(gn1_w), r2(gn1_b), r2(gn1_a),
                    h0, W_lin.T, r2(b_lin))
    return out[:N]


# speedup vs baseline: 17.2077x; 3.5341x over previous
import functools

import jax
import jax.numpy as jnp
from jax import lax
from jax.experimental import pallas as pl
from jax.experimental.pallas import tpu as pltpu
from jax.experimental.pallas import tpu_sc as plsc

N = 10000
E = 320000
D = 128
G = 16

NP = 10240
BN = 1024
NB = NP // BN
NC = 2
NS = 16
NW = NC * NS
K = 128
EW = 10240
NCH = EW // K
EP = NW * EW
RPT = NP // NS
DW = 128
EPS = 1e-5
HI = lax.Precision.HIGHEST


def _sc_scatter_body(y_hbm, idx_hbm, zeros_hbm, z_hbm,
                     ibuf0, ibuf1, buf0, buf1, z_sh,
                     isem0, isem1, sem0, sem1):
    c = lax.axis_index("c")
    s = lax.axis_index("s")
    wid = s * NC + c
    ibufs = (ibuf0, ibuf1)
    isems = (isem0, isem1)
    bufs = (buf0, buf1)
    sems = (sem0, sem1)

    def idx_start(j, b):
        pltpu.async_copy(idx_hbm.at[wid, j], ibufs[b], isems[b])

    def idx_wait(j, b):
        pltpu.make_async_copy(idx_hbm.at[wid, j], ibufs[b], isems[b]).wait()

    def gather_start(b):
        pltpu.async_copy(y_hbm.at[ibufs[b].at[0]], bufs[b], sems[b])

    def gather_wait(b):
        pltpu.make_async_copy(y_hbm.at[ibufs[b].at[0]], bufs[b],
                              sems[b]).wait()

    def scatter(b):
        pltpu.sync_copy(bufs[b], z_sh.at[ibufs[b].at[1]], add=True)

    pltpu.sync_copy(zeros_hbm.at[pl.ds(s * RPT, RPT)],
                    z_sh.at[pl.ds(s * RPT, RPT)])
    plsc.subcore_barrier()

    idx_start(0, 0)
    idx_start(1, 1)
    idx_wait(0, 0)
    gather_start(0)

    @pl.loop(0, NCH - 2, step=2)
    def _(j):
        for b in range(2):
            jb = j + b
            ob = 1 - b
            idx_wait(jb + 1, ob)
            gather_start(ob)
            gather_wait(b)
            scatter(b)
            idx_start(jb + 2, b)

    b = (NCH - 2) % 2
    idx_wait(NCH - 1, 1 - b)
    gather_start(1 - b)
    gather_wait(b)
    scatter(b)
    gather_wait(1 - b)
    scatter(1 - b)

    plsc.subcore_barrier()
    pltpu.sync_copy(z_sh.at[pl.ds(s * RPT, RPT)],
                    z_hbm.at[c, pl.ds(s * RPT, RPT)])


@functools.lru_cache(maxsize=None)
def _sc_scatter():
    return pl.kernel(
        _sc_scatter_body,
        out_type=jax.ShapeDtypeStruct((NC, NP, D), jnp.float32),
        mesh=plsc.VectorSubcoreMesh(core_axis_name="c", subcore_axis_name="s",
                                    num_cores=NC, num_subcores=NS),
        scratch_types=[
            pltpu.VMEM((2, K), jnp.int32),
            pltpu.VMEM((2, K), jnp.int32),
            pltpu.VMEM((K, D), jnp.float32),
            pltpu.VMEM((K, D), jnp.float32),
            pltpu.VMEM_SHARED((NP, D), jnp.float32),
            pltpu.SemaphoreType.DMA,
            pltpu.SemaphoreType.DMA,
            pltpu.SemaphoreType.DMA,
            pltpu.SemaphoreType.DMA,
        ],
    )


def _sc_deg_body(col_hbm, zeros_hbm, ones_hbm, deg_hbm,
                 col_v, ones_v, deg_sh):
    c = lax.axis_index("c")
    s = lax.axis_index("s")
    wid = s * NC + c
    pltpu.sync_copy(col_hbm.at[wid], col_v)
    pltpu.sync_copy(ones_hbm, ones_v)
    pltpu.sync_copy(zeros_hbm.at[pl.ds(s * RPT, RPT)],
                    deg_sh.at[pl.ds(s * RPT, RPT)])
    plsc.subcore_barrier()

    @pl.loop(0, NCH)
    def _(j):
        pltpu.sync_copy(ones_v, deg_sh.at[col_v.at[j]], add=True)

    plsc.subcore_barrier()
    pltpu.sync_copy(deg_sh.at[pl.ds(s * RPT, RPT)],
                    deg_hbm.at[c, pl.ds(s * RPT, RPT)])


@functools.lru_cache(maxsize=None)
def _sc_deg():
    return pl.kernel(
        _sc_deg_body,
        out_type=jax.ShapeDtypeStruct((NC, NP, DW), jnp.float32),
        mesh=plsc.VectorSubcoreMesh(core_axis_name="c", subcore_axis_name="s",
                                    num_cores=NC, num_subcores=NS),
        scratch_types=[
            pltpu.VMEM((NCH, K), jnp.int32),
            pltpu.VMEM((K, DW), jnp.float32),
            pltpu.VMEM_SHARED((NP, DW), jnp.float32),
        ],
    )


def _dinv(deg_ref):
    deg = deg_ref[0, :, 0] + deg_ref[1, :, 0] + 1.0
    return lax.rsqrt(deg)


def _tc_y_body(x_ref, w_ref, deg_ref, y_ref):
    xw = jnp.dot(x_ref[...], w_ref[...], preferred_element_type=jnp.float32)
    y_ref[...] = xw * _dinv(deg_ref)[:, None]


def _seg_stats(bvec, t):
    oh = (bvec[None, :] == lax.broadcasted_iota(jnp.int32, (G, BN), 0))
    oh = oh.astype(jnp.float32)
    s1 = jnp.dot(oh, t, preferred_element_type=jnp.float32, precision=HI)
    cnt = jnp.broadcast_to(jnp.sum(oh, axis=1)[:, None], (G, D))
    return s1, cnt


def _tc_t_body(z_ref, y_ref, deg_ref, b0_ref, batch_ref, t_ref, stats_ref):
    i = pl.program_id(0)
    dinv = _dinv(deg_ref)
    t = dinv[:, None] * (z_ref[0] + z_ref[1] + y_ref[...]) + b0_ref[...]
    t_ref[...] = t
    s1, cnt = _seg_stats(batch_ref[0, :], t)
    blk = jnp.stack([s1, cnt])

    @pl.when(i == 0)
    def _():
        stats_ref[...] = blk

    @pl.when(i > 0)
    def _():
        stats_ref[...] = stats_ref[...] + blk


def _tc_var_body(t_ref, stats_ref, batch_ref, a_ref, var_ref):
    i = pl.program_id(0)
    st = stats_ref[...]
    cnt = jnp.maximum(st[1], 1.0)
    mean_a = st[0] / cnt * a_ref[...]
    bvec = batch_ref[0, :]
    oh_gn = (bvec[None, :] == lax.broadcasted_iota(jnp.int32, (G, BN), 0))
    oh_gn = oh_gn.astype(jnp.float32)
    oh_ng = (bvec[:, None] == lax.broadcasted_iota(jnp.int32, (BN, G), 1))
    oh_ng = oh_ng.astype(jnp.float32)
    dlt = t_ref[...] - jnp.dot(oh_ng, mean_a,
                               preferred_element_type=jnp.float32,
                               precision=HI)
    blk = jnp.dot(oh_gn, dlt * dlt, preferred_element_type=jnp.float32,
                  precision=HI)

    @pl.when(i == 0)
    def _():
        var_ref[...] = blk

    @pl.when(i > 0)
    def _():
        var_ref[...] = var_ref[...] + blk


def _norm_factors(s1, vsum, cnt, a):
    cnt = jnp.maximum(cnt, 1.0)
    mean = s1 / cnt
    invstd = lax.rsqrt(vsum / cnt + EPS)
    return mean * a, invstd


def _row_norm(bvec, t, mean_a, invstd, w, b):
    oh = (bvec[:, None] == lax.broadcasted_iota(jnp.int32, (BN, G), 1))
    oh = oh.astype(jnp.float32)
    mean_rows = jnp.dot(oh, mean_a, preferred_element_type=jnp.float32,
                        precision=HI)
    invstd_rows = jnp.dot(oh, invstd, preferred_element_type=jnp.float32,
                          precision=HI)
    return w * ((t - mean_rows) * invstd_rows) + b


def _tc_h0_body(t_ref, stats_ref, var_ref, batch_ref, w_ref, b_ref, a_ref,
                ggw_ref, h_ref, m_ref):
    st = stats_ref[...]
    mean_a, invstd = _norm_factors(st[0], var_ref[...], st[1], a_ref[...])
    h0 = _row_norm(batch_ref[0, :], t_ref[...], mean_a, invstd,
                   w_ref[...], b_ref[...])
    h_ref[...] = h0
    m_ref[...] = jnp.dot(h0, ggw_ref[...], preferred_element_type=jnp.float32)


def _gru(z_ref, h_ref, wih_ref, whh_ref, bih_ref, bhh_ref):
    agg = z_ref[0] + z_ref[1]
    h = h_ref[...]
    gi = jnp.dot(agg, wih_ref[...],
                 preferred_element_type=jnp.float32) + bih_ref[...]
    gh = jnp.dot(h, whh_ref[...],
                 preferred_element_type=jnp.float32) + bhh_ref[...]
    r = jax.nn.sigmoid(gi[:, :D] + gh[:, :D])
    zz = jax.nn.sigmoid(gi[:, D:2 * D] + gh[:, D:2 * D])
    n = jnp.tanh(gi[:, 2 * D:] + r * gh[:, 2 * D:])
    return (1.0 - zz) * n + zz * h


def _tc_gru_body(z_ref, h_ref, wih_ref, whh_ref, bih_ref, bhh_ref, wn_ref,
                 hn_ref, m_ref):
    hn = _gru(z_ref, h_ref, wih_ref, whh_ref, bih_ref, bhh_ref)
    hn_ref[...] = hn
    m_ref[...] = jnp.dot(hn, wn_ref[...], preferred_element_type=jnp.float32)


def _tc_gru2_body(z_ref, h_ref, wih_ref, whh_ref, bih_ref, bhh_ref, batch_ref,
                  u_ref, stats_ref):
    i = pl.program_id(0)
    hn = _gru(z_ref, h_ref, wih_ref, whh_ref, bih_ref, bhh_ref)
    u = 0.5 * hn * (1.0 + lax.erf(hn * 0.7071067811865476))
    u_ref[...] = u
    s1, cnt = _seg_stats(batch_ref[0, :], u)
    blk = jnp.stack([s1, cnt])

    @pl.when(i == 0)
    def _():
        stats_ref[...] = blk

    @pl.when(i > 0)
    def _():
        stats_ref[...] = stats_ref[...] + blk


def _tc_fin_body(u_ref, stats_ref, var_ref, batch_ref, w_ref, b_ref, a_ref,
                 h0_ref, wlin_ref, blin_ref, out_ref):
    st = stats_ref[...]
    mean_a, invstd = _norm_factors(st[0], var_ref[...], st[1], a_ref[...])
    gn = _row_norm(batch_ref[0, :], u_ref[...], mean_a, invstd,
                   w_ref[...], b_ref[...])
    v = h0_ref[...] + gn
    out_ref[...] = jnp.dot(v, wlin_ref[...],
                           preferred_element_type=jnp.float32) + blin_ref[...]


_ROW = pl.BlockSpec((BN, D), lambda i: (i, 0))
_ROW3 = pl.BlockSpec((BN, 3 * D), lambda i: (i, 0))
_ZPART = pl.BlockSpec((NC, BN, D), lambda i: (0, i, 0))
_DEG = pl.BlockSpec((NC, BN, DW), lambda i: (0, i, 0))
_BATCH = pl.BlockSpec((1, BN), lambda i: (0, i))
_FULL = lambda *shape: pl.BlockSpec(shape, lambda i: (0,) * len(shape))

_f32 = jnp.float32


def _call_y(x_p, W0, deg):
    return pl.pallas_call(
        _tc_y_body,
        grid=(NB,),
        in_specs=[_ROW, _FULL(D, D), _DEG],
        out_specs=_ROW,
        out_shape=jax.ShapeDtypeStruct((NP, D), _f32),
    )(x_p, W0, deg)


def _call_t(z, y, deg, b0, batch_p):
    return pl.pallas_call(
        _tc_t_body,
        grid=(NB,),
        in_specs=[_ZPART, _ROW, _DEG, _FULL(1, D), _BATCH],
        out_specs=[_ROW, _FULL(2, G, D)],
        out_shape=[jax.ShapeDtypeStruct((NP, D), _f32),
                   jax.ShapeDtypeStruct((2, G, D), _f32)],
    )(z, y, deg, b0, batch_p)


def _call_var(t, stats, batch_p, a):
    return pl.pallas_call(
        _tc_var_body,
        grid=(NB,),
        in_specs=[_ROW, _FULL(2, G, D), _BATCH, _FULL(1, D)],
        out_specs=_FULL(G, D),
        out_shape=jax.ShapeDtypeStruct((G, D), _f32),
    )(t, stats, batch_p, a)


def _call_h0(t, stats, var, batch_p, w, b, a, ggw):
    return pl.pallas_call(
        _tc_h0_body,
        grid=(NB,),
        in_specs=[_ROW, _FULL(2, G, D), _FULL(G, D), _BATCH, _FULL(1, D),
                  _FULL(1, D), _FULL(1, D), _FULL(D, D)],
        out_specs=[_ROW, _ROW],
        out_shape=[jax.ShapeDtypeStruct((NP, D), _f32),
                   jax.ShapeDtypeStruct((NP, D), _f32)],
    )(t, stats, var, batch_p, w, b, a, ggw)


def _call_gru(z, h, wih_t, whh_t, bih, bhh, wn):
    return pl.pallas_call(
        _tc_gru_body,
        grid=(NB,),
        in_specs=[_ZPART, _ROW, _FULL(D, 3 * D), _FULL(D, 3 * D),
                  _FULL(1, 3 * D), _FULL(1, 3 * D), _FULL(D, D)],
        out_specs=[_ROW, _ROW],
        out_shape=[jax.ShapeDtypeStruct((NP, D), _f32),
                   jax.ShapeDtypeStruct((NP, D), _f32)],
    )(z, h, wih_t, whh_t, bih, bhh, wn)


def _call_gru2(z, h, wih_t, whh_t, bih, bhh, batch_p):
    return pl.pallas_call(
        _tc_gru2_body,
        grid=(NB,),
        in_specs=[_ZPART, _ROW, _FULL(D, 3 * D), _FULL(D, 3 * D),
                  _FULL(1, 3 * D), _FULL(1, 3 * D), _BATCH],
        out_specs=[_ROW, _FULL(2, G, D)],
        out_shape=[jax.ShapeDtypeStruct((NP, D), _f32),
                   jax.ShapeDtypeStruct((2, G, D), _f32)],
    )(z, h, wih_t, whh_t, bih, bhh, batch_p)


def _call_fin(u, stats2, var2, batch_p, w, b, a, h0, wlin_t, blin):
    return pl.pallas_call(
        _tc_fin_body,
        grid=(NB,),
        in_specs=[_ROW, _FULL(2, G, D), _FULL(G, D), _BATCH, _FULL(1, D),
                  _FULL(1, D), _FULL(1, D), _ROW, _FULL(D, D), _FULL(1, D)],
        out_specs=_ROW,
        out_shape=jax.ShapeDtypeStruct((NP, D), _f32),
    )(u, stats2, var2, batch_p, w, b, a, h0, wlin_t, blin)


def kernel(x, edge_index, batch, W0, b0, gn0_w, gn0_b, gn0_a, ggc_w,
           gru_wih, gru_whh, gru_bih, gru_bhh, gn1_w, gn1_b, gn1_a,
           W_lin, b_lin):
    x_p = jnp.zeros((NP, D), _f32).at[:N].set(x)
    batch_p = jnp.full((1, NP), G, jnp.int32).at[0, :N].set(batch)

    pad = EP - E
    dummy = N + jnp.arange(pad, dtype=jnp.int32) % (NP - N)
    row = jnp.concatenate([edge_index[0], dummy]).reshape(NW, NCH, 1, K)
    col = jnp.concatenate([edge_index[1], dummy]).reshape(NW, NCH, 1, K)
    idx = jnp.concatenate([row, col], axis=2)
    col3 = col.reshape(NW, NCH, K)

    zeros_d = jnp.zeros((NP, D), _f32)
    ones_w = jnp.ones((K, DW), _f32)

    r2 = lambda v: v.reshape(1, -1)
    wih_t = gru_wih.T
    whh_t = gru_whh.T

    deg = _sc_deg()(col3, zeros_d, ones_w)
    y = _call_y(x_p, W0, deg)
    z = _sc_scatter()(y, idx, zeros_d)
    t, stats = _call_t(z, y, deg, r2(b0), batch_p)
    var0 = _call_var(t, stats, batch_p, r2(gn0_a))
    h0, m0 = _call_h0(t, stats, var0, batch_p, r2(gn0_w), r2(gn0_b),
                      r2(gn0_a), ggc_w[0])
    a0 = _sc_scatter()(m0, idx, zeros_d)
    h1, m1 = _call_gru(a0, h0, wih_t, whh_t, r2(gru_bih), r2(gru_bhh),
                       ggc_w[1])
    a1 = _sc_scatter()(m1, idx, zeros_d)
    u, stats2 = _call_gru2(a1, h1, wih_t, whh_t, r2(gru_bih), r2(gru_bhh),
                           batch_p)
    var1 = _call_var(u, stats2, batch_p, r2(gn1_a))
    out = _call_fin(u, stats2, var1, batch_p, r2(gn1_w), r2(gn1_b),
                    r2(gn1_a), h0, W_lin.T, r2(b_lin))
    return out[:N]
